# baseline (speedup 1.0000x reference)
import functools

import jax
import jax.numpy as jnp
from jax import lax
from jax.experimental import pallas as pl
from jax.experimental.pallas import tpu as pltpu

_EPS_SQ = 1e-24


def _pool_kernel(x_ref, out_ref, fb_ref, *, inv_hw):
    ph = pl.program_id(1)

    @pl.when(ph == 0)
    def _():
        x = x_ref[0]
        sumsq = jnp.sum(x * x, axis=-1, keepdims=True)
        scale = lax.rsqrt(jnp.maximum(sumsq, _EPS_SQ))
        mean = jnp.sum(x, axis=-1, keepdims=True) * inv_hw
        ch_sumsq = jnp.sum(mean * mean, axis=0, keepdims=True)
        fb_ref[...] = mean * lax.rsqrt(jnp.maximum(ch_sumsq, _EPS_SQ))
        out_ref[0, 0] = x * scale

    @pl.when(ph == 1)
    def _():
        out_ref[0, 0] = jnp.broadcast_to(fb_ref[...], out_ref.shape[2:])


def kernel(x):
    N, C, H, W = x.shape
    HW = H * W
    x3 = x.reshape(N, C, HW)

    out = pl.pallas_call(
        functools.partial(_pool_kernel, inv_hw=1.0 / HW),
        out_shape=jax.ShapeDtypeStruct((N, 2, C, HW), x.dtype),
        grid=(N, 2),
        in_specs=[pl.BlockSpec((1, C, HW), lambda n, p: (n, 0, 0))],
        out_specs=pl.BlockSpec((1, 1, C, HW), lambda n, p: (n, p, 0, 0)),
        scratch_shapes=[pltpu.VMEM((C, 1), jnp.float32)],
        compiler_params=pltpu.CompilerParams(
            dimension_semantics=("parallel", "arbitrary"),
            vmem_limit_bytes=48 * 1024 * 1024,
        ),
        cost_estimate=pl.CostEstimate(
            flops=6 * N * C * HW,
            transcendentals=2 * N * C,
            bytes_accessed=3 * N * C * HW * x.dtype.itemsize,
        ),
    )(x3)
    return out.reshape(N, 2 * C, H, W)


# speedup vs baseline: 1.1417x; 1.1417x over previous
import functools

import jax
import jax.numpy as jnp
from jax import lax
from jax.experimental import pallas as pl
from jax.experimental.pallas import tpu as pltpu

_EPS_SQ = 1e-24


def _pool_kernel(x_ref, out_ref, fb_ref, *, inv_hw):
    ph = pl.program_id(1)

    @pl.when(ph == 0)
    def _():
        x = x_ref[0]
        sh = jnp.sum(x, axis=1)
        ssh = jnp.sum(x * x, axis=1)
        s = jnp.sum(sh, axis=-1, keepdims=True)
        ss = jnp.sum(ssh, axis=-1, keepdims=True)
        scale = lax.rsqrt(jnp.maximum(ss, _EPS_SQ))
        mean = s * inv_hw
        ch = jnp.sum(mean * mean, axis=0, keepdims=True)
        fb_ref[...] = mean * lax.rsqrt(jnp.maximum(ch, _EPS_SQ))
        out_ref[0] = x * scale[:, :, None]

    @pl.when(ph == 1)
    def _():
        out_ref[0] = jnp.broadcast_to(fb_ref[...][:, :, None],
                                      out_ref.shape[1:])


def kernel(x):
    N, C, H, W = x.shape

    return pl.pallas_call(
        functools.partial(_pool_kernel, inv_hw=1.0 / (H * W)),
        out_shape=jax.ShapeDtypeStruct((N, 2 * C, H, W), x.dtype),
        grid=(N, 2),
        in_specs=[pl.BlockSpec((1, C, H, W), lambda n, p: (n, 0, 0, 0))],
        out_specs=pl.BlockSpec((1, C, H, W), lambda n, p: (n, p, 0, 0)),
        scratch_shapes=[pltpu.VMEM((C, 1), jnp.float32)],
        compiler_params=pltpu.CompilerParams(
            dimension_semantics=("parallel", "arbitrary"),
            vmem_limit_bytes=50 * 1024 * 1024,
        ),
        cost_estimate=pl.CostEstimate(
            flops=6 * N * C * H * W,
            transcendentals=2 * N * C,
            bytes_accessed=3 * N * C * H * W * x.dtype.itemsize,
        ),
    )(x)


# speedup vs baseline: 5.8327x; 5.1090x over previous
import functools

import jax
import jax.numpy as jnp
from jax import lax
from jax.experimental import pallas as pl
from jax.experimental.pallas import tpu as pltpu

_EPS_SQ = 1e-24


def _pool_kernel(x_ref, out_ref, fb_ref, *, inv_hw):
    ph = pl.program_id(1)
    h, w, c = x_ref.shape[1:]

    @pl.when(ph == 0)
    def _():
        x = x_ref[0].reshape(h * w, c)
        ss = jnp.sum(x * x, axis=0, keepdims=True)
        s = jnp.sum(x, axis=0, keepdims=True)
        scale = lax.rsqrt(jnp.maximum(ss, _EPS_SQ))
        mean = s * inv_hw
        ch = jnp.sum(mean * mean, axis=1, keepdims=True)
        fb_ref[...] = mean * lax.rsqrt(jnp.maximum(ch, _EPS_SQ))
        out_ref[0] = (x * scale).reshape(h, w, c)

    @pl.when(ph == 1)
    def _():
        out_ref[0] = jnp.broadcast_to(fb_ref[...], (h * w, c)).reshape(h, w, c)


def kernel(x):
    N, C, H, W = x.shape

    xt = jnp.transpose(x, (0, 2, 3, 1))
    out = pl.pallas_call(
        functools.partial(_pool_kernel, inv_hw=1.0 / (H * W)),
        out_shape=jax.ShapeDtypeStruct((N, H, W, 2 * C), x.dtype),
        grid=(N, 2),
        in_specs=[pl.BlockSpec((1, H, W, C), lambda n, p: (n, 0, 0, 0))],
        out_specs=pl.BlockSpec((1, H, W, C), lambda n, p: (n, 0, 0, p)),
        scratch_shapes=[pltpu.VMEM((1, C), jnp.float32)],
        compiler_params=pltpu.CompilerParams(
            dimension_semantics=("parallel", "arbitrary"),
            vmem_limit_bytes=50 * 1024 * 1024,
        ),
        cost_estimate=pl.CostEstimate(
            flops=6 * N * C * H * W,
            transcendentals=2 * N * C,
            bytes_accessed=3 * N * C * H * W * x.dtype.itemsize,
        ),
    )(xt)
    return jnp.transpose(out, (0, 3, 1, 2))


# speedup vs baseline: 8.0982x; 1.3884x over previous
import functools

import jax
import jax.numpy as jnp
from jax import lax
from jax.experimental import pallas as pl
from jax.experimental.pallas import tpu as pltpu

_EPS_SQ = 1e-24


def _pool_kernel(x_ref, out_ref, *, inv_hw):
    h, w, c = x_ref.shape[1:]
    x = x_ref[0].reshape(h * w, c)
    ss = jnp.sum(x * x, axis=0, keepdims=True)
    s = jnp.sum(x, axis=0, keepdims=True)
    scale = lax.rsqrt(jnp.maximum(ss, _EPS_SQ))
    mean = s * inv_hw
    ch = jnp.sum(mean * mean, axis=1, keepdims=True)
    fb = mean * lax.rsqrt(jnp.maximum(ch, _EPS_SQ))
    out_ref[0, :, :, 0:c] = (x * scale).reshape(h, w, c)
    out_ref[0, :, :, c:2 * c] = jnp.broadcast_to(fb, (h * w, c)).reshape(
        h, w, c)


def kernel(x):
    N, C, H, W = x.shape

    xt = jnp.transpose(x, (0, 2, 3, 1))
    out = pl.pallas_call(
        functools.partial(_pool_kernel, inv_hw=1.0 / (H * W)),
        out_shape=jax.ShapeDtypeStruct((N, H, W, 2 * C), x.dtype),
        grid=(N,),
        in_specs=[pl.BlockSpec((1, H, W, C), lambda n: (n, 0, 0, 0))],
        out_specs=pl.BlockSpec((1, H, W, 2 * C), lambda n: (n, 0, 0, 0)),
        compiler_params=pltpu.CompilerParams(
            dimension_semantics=("parallel",),
            vmem_limit_bytes=50 * 1024 * 1024,
        ),
        cost_estimate=pl.CostEstimate(
            flops=6 * N * C * H * W,
            transcendentals=2 * N * C,
            bytes_accessed=3 * N * C * H * W * x.dtype.itemsize,
        ),
    )(xt)
    return jnp.transpose(out, (0, 3, 1, 2))


# speedup vs baseline: 8.3044x; 1.0255x over previous
import functools

import jax
import jax.numpy as jnp
from jax import lax
from jax.experimental import pallas as pl
from jax.experimental.pallas import tpu as pltpu

_EPS_SQ = 1e-24


def _pool_kernel(x_ref, out_ref, *, inv_hw):
    tb, h, w, c = x_ref.shape
    for b in range(tb):
        x = x_ref[b].reshape(h * w, c)
        ss = jnp.sum(x * x, axis=0, keepdims=True)
        s = jnp.sum(x, axis=0, keepdims=True)
        scale = lax.rsqrt(jnp.maximum(ss, _EPS_SQ))
        mean = s * inv_hw
        ch = jnp.sum(mean * mean, axis=1, keepdims=True)
        fb = mean * lax.rsqrt(jnp.maximum(ch, _EPS_SQ))
        out_ref[b, :, :, 0:c] = (x * scale).reshape(h, w, c)
        out_ref[b, :, :, c:2 * c] = jnp.broadcast_to(fb, (h * w, c)).reshape(
            h, w, c)


def kernel(x):
    N, C, H, W = x.shape

    xt = jnp.transpose(x, (0, 2, 3, 1))
    out = pl.pallas_call(
        functools.partial(_pool_kernel, inv_hw=1.0 / (H * W)),
        out_shape=jax.ShapeDtypeStruct((N, H, W, 2 * C), x.dtype),
        grid=(N // 2,),
        in_specs=[pl.BlockSpec((2, H, W, C), lambda n: (n, 0, 0, 0))],
        out_specs=pl.BlockSpec((2, H, W, 2 * C), lambda n: (n, 0, 0, 0)),
        compiler_params=pltpu.CompilerParams(
            dimension_semantics=("parallel",),
            vmem_limit_bytes=50 * 1024 * 1024,
        ),
        cost_estimate=pl.CostEstimate(
            flops=6 * N * C * H * W,
            transcendentals=2 * N * C,
            bytes_accessed=3 * N * C * H * W * x.dtype.itemsize,
        ),
    )(xt)
    return jnp.transpose(out, (0, 3, 1, 2))


# speedup vs baseline: 8.3045x; 1.0000x over previous
import functools

import jax
import jax.numpy as jnp
from jax import lax
from jax.experimental import pallas as pl
from jax.experimental.pallas import tpu as pltpu

_EPS_SQ = 1e-24


def _pool_kernel(x_ref, out_ref, *, inv_hw):
    tb, h, w, c = x_ref.shape
    for b in range(tb):
        x = x_ref[b].reshape(h * w, c)
        ss = jnp.sum(x * x, axis=0, keepdims=True)
        s = jnp.sum(x, axis=0, keepdims=True)
        scale = lax.rsqrt(jnp.maximum(ss, _EPS_SQ))
        mean = s * inv_hw
        ch = jnp.sum(mean * mean, axis=1, keepdims=True)
        fb = mean * lax.rsqrt(jnp.maximum(ch, _EPS_SQ))
        out_ref[b, :, :, 0:c] = (x * scale).reshape(h, w, c)
        out_ref[b, :, :, c:2 * c] = jnp.broadcast_to(fb, (h * w, c)).reshape(
            h, w, c)


def kernel(x):
    N, C, H, W = x.shape

    tb = 2 if N % 2 == 0 else 1

    xt = jnp.transpose(x, (0, 2, 3, 1))
    out = pl.pallas_call(
        functools.partial(_pool_kernel, inv_hw=1.0 / (H * W)),
        out_shape=jax.ShapeDtypeStruct((N, H, W, 2 * C), x.dtype),
        grid=(N // tb,),
        in_specs=[pl.BlockSpec((tb, H, W, C), lambda n: (n, 0, 0, 0))],
        out_specs=pl.BlockSpec((tb, H, W, 2 * C), lambda n: (n, 0, 0, 0)),
        compiler_params=pltpu.CompilerParams(
            dimension_semantics=("parallel",),
            vmem_limit_bytes=50 * 1024 * 1024,
        ),
        cost_estimate=pl.CostEstimate(
            flops=6 * N * C * H * W,
            transcendentals=2 * N * C,
            bytes_accessed=3 * N * C * H * W * x.dtype.itemsize,
        ),
    )(xt)
    return jnp.transpose(out, (0, 3, 1, 2))
